# baseline (speedup 1.0000x reference)
import jax
import jax.numpy as jnp
from jax import lax
from jax.experimental import pallas as pl
from jax.experimental.pallas import tpu as pltpu
from jax.experimental.pallas import tpu_sc as plsc

_VOCAB = 1000000
_EMBED = 32
_WORDS = 50
_BATCH = 4096

_VQ = 512
_CH = 8
_BB = _VQ * _CH
_TCNB = 245
_VPAD = _TCNB * _BB


def _detile_body(x_ref, o_ref, scr):
    for c in range(_CH):
        for q in range(4):
            xq = x_ref[:, c * _VQ + 128 * q: c * _VQ + 128 * (q + 1)]
            scr[c, :, 32 * q:32 * (q + 1)] = xq.astype(jnp.bfloat16).T
        o_ref[pl.ds(c * _VQ * _EMBED, _VQ * _EMBED)] = (
            scr[c].reshape(_VQ * _EMBED))


def _detile(table_t):
    return pl.pallas_call(
        _detile_body,
        grid=(_TCNB,),
        in_specs=[pl.BlockSpec((_EMBED, _BB), lambda j: (0, j))],
        out_specs=pl.BlockSpec((_BB * _EMBED,), lambda j: (j,)),
        out_shape=jax.ShapeDtypeStruct((_VPAD * _EMBED,), jnp.bfloat16),
        scratch_shapes=[pltpu.VMEM((_CH, 128, 128), jnp.bfloat16)],
    )(table_t)


_NC, _NS = 2, 16
_NW = _NC * _NS
_BPW = _BATCH // _NW
_GL = 16
_PAIRB = 2 * _GL
_NP = _BPW // _PAIRB
_RPP = _PAIRB * _WORDS
_IPT = _NP * _RPP
_IDXW = 128
_ICH = 16


def _rsqrt(x):
    i = plsc.bitcast(x, jnp.int32)
    i = jnp.int32(0x5F3759DF) - lax.shift_right_logical(i, 1)
    y = plsc.bitcast(i, jnp.float32)
    for _ in range(3):
        y = y * (1.5 - 0.5 * x * y * y)
    return y


def _tile_body(vid_hbm, table_hbm, f0_hbm, f1_hbm, fb0_hbm, fb1_hbm, out_hbm,
               idx_f, rows_v, f0_v, f1_v, fb0_v, fb1_v, o0_v, o1_v,
               sem0, sem1):
    wid = lax.axis_index("s") * _NC + lax.axis_index("c")
    sems = (sem0, sem1)

    stage = [
        pltpu.async_copy(vid_hbm.at[pl.ds(wid * _IPT, _IPT)], idx_f, sem0),
        pltpu.async_copy(f0_hbm, f0_v, sem0),
        pltpu.async_copy(f1_hbm, f1_v, sem0),
        pltpu.async_copy(fb0_hbm, fb0_v, sem0),
        pltpu.async_copy(fb1_hbm, fb1_v, sem0),
    ]
    for c in stage:
        c.wait()

    def idx_body(i, carry):
        s = i * _ICH
        v = idx_f[pl.ds(s, _ICH)]
        rowid = ((v & jnp.int32(~511))
                 + lax.shift_left(v & jnp.int32(127), 2)
                 + (lax.shift_right_logical(v, 7) & jnp.int32(3)))
        idx_f[pl.ds(s, _ICH)] = rowid
        return carry

    lax.fori_loop(0, _IPT // _ICH, idx_body, jnp.int32(0))

    lanes = lax.iota(jnp.int32, _GL)
    bias0 = fb0_v[...]
    bias1 = fb1_v[...]
    row_base = lanes * _WORDS
    for g in range(_BPW // _GL):
        o0_v[pl.ds(g * _GL, _GL)] = bias0
        o1_v[pl.ds(g * _GL, _GL)] = bias1

    def fire(p):
        buf = p % 2
        sem = sems[buf]
        copies = []
        dst = 0
        while dst < _RPP:
            n = min(_IDXW, _RPP - dst)
            copies.append(pltpu.async_copy(
                table_hbm.at[idx_f.at[pl.ds(p * _RPP + dst, n)]],
                rows_v.at[buf, pl.ds(dst, n)], sem))
            dst += n
        return copies

    pending = {0: fire(0)}
    for p in range(_NP):
        buf = p % 2
        for c in pending.pop(p):
            c.wait()
        if p + 1 < _NP:
            pending[p + 1] = fire(p + 1)

        def w_body(w, carry, _rb=row_base, _buf=buf, _p=p):
            rowA = _rb + w
            rowB = rowA + _GL * _WORDS
            cw = jnp.full((_GL,), w * _EMBED, jnp.int32)
            z = jnp.zeros((_GL,), jnp.float32)
            ssA, a0A, a1A = z, z, z
            ssB, a0B, a1B = z, z, z
            himask = jnp.int32(-65536)
            for jp in range(_EMBED // 2):
                colv = jnp.full((_GL,), jp, jnp.int32)
                cidx = cw + 2 * jp
                c0e = plsc.load_gather(f0_v, [cidx])
                c1e = plsc.load_gather(f1_v, [cidx])
                c0o = plsc.load_gather(f0_v, [cidx + 1])
                c1o = plsc.load_gather(f1_v, [cidx + 1])
                wA = plsc.load_gather(rows_v.at[_buf], [rowA, colv])
                wB = plsc.load_gather(rows_v.at[_buf], [rowB, colv])
                dAe = plsc.bitcast(lax.shift_left(wA, 16), jnp.float32)
                dAo = plsc.bitcast(wA & himask, jnp.float32)
                dBe = plsc.bitcast(lax.shift_left(wB, 16), jnp.float32)
                dBo = plsc.bitcast(wB & himask, jnp.float32)
                ssA = ssA + dAe * dAe + dAo * dAo
                a0A = a0A + dAe * c0e + dAo * c0o
                a1A = a1A + dAe * c1e + dAo * c1o
                ssB = ssB + dBe * dBe + dBo * dBo
                a0B = a0B + dBe * c0e + dBo * c0o
                a1B = a1B + dBe * c1e + dBo * c1o
            sA = _rsqrt(jnp.maximum(ssA, 1.0))
            sB = _rsqrt(jnp.maximum(ssB, 1.0))
            plsc.addupdate(o0_v.at[pl.ds(_p * _PAIRB, _GL)], sA * a0A)
            plsc.addupdate(o1_v.at[pl.ds(_p * _PAIRB, _GL)], sA * a1A)
            plsc.addupdate(o0_v.at[pl.ds(_p * _PAIRB + _GL, _GL)], sB * a0B)
            plsc.addupdate(o1_v.at[pl.ds(_p * _PAIRB + _GL, _GL)], sB * a1B)
            return carry

        lax.fori_loop(0, _WORDS, w_body, jnp.int32(0))

    pltpu.sync_copy(o0_v, out_hbm.at[0, pl.ds(wid * _BPW, _BPW)])
    pltpu.sync_copy(o1_v, out_hbm.at[1, pl.ds(wid * _BPW, _BPW)])


def _sc_logits(vid1d, table, f0, f1, fb0, fb1):
    mesh = plsc.VectorSubcoreMesh(core_axis_name="c", subcore_axis_name="s")
    return pl.kernel(
        _tile_body,
        out_type=jax.ShapeDtypeStruct((2, _BATCH), jnp.float32),
        mesh=mesh,
        compiler_params=pltpu.CompilerParams(
            needs_layout_passes=False, use_tc_tiling_on_sc=False),
        scratch_types=[
            pltpu.VMEM((_IPT,), jnp.int32),
            pltpu.VMEM((2, _RPP, _EMBED // 2), jnp.int32),
            pltpu.VMEM((_WORDS * _EMBED,), jnp.float32),
            pltpu.VMEM((_WORDS * _EMBED,), jnp.float32),
            pltpu.VMEM((_GL,), jnp.float32),
            pltpu.VMEM((_GL,), jnp.float32),
            pltpu.VMEM((_BPW,), jnp.float32),
            pltpu.VMEM((_BPW,), jnp.float32),
            pltpu.SemaphoreType.DMA,
            pltpu.SemaphoreType.DMA,
        ],
    )(vid1d, table, f0, f1, fb0, fb1)


@jax.jit
def _run(vocab_id, table, fc_w, fc_b):
    lin = _detile(table.T)
    table_lin = lax.bitcast_convert_type(
        lin.reshape(_VPAD, _EMBED // 2, 2), jnp.int32)
    vid1d = vocab_id.reshape(-1)
    fb0 = jnp.full((_GL,), fc_b[0], jnp.float32)
    fb1 = jnp.full((_GL,), fc_b[1], jnp.float32)
    out2 = _sc_logits(vid1d, table_lin, fc_w[0], fc_w[1], fb0, fb1)
    return out2.T


def kernel(vocab_id, table, fc_w, fc_b):
    return _run(vocab_id, table, fc_w, fc_b)


# speedup vs baseline: 66.3759x; 66.3759x over previous
import jax
import jax.numpy as jnp
from jax import lax
from jax.experimental import pallas as pl
from jax.experimental.pallas import tpu as pltpu
from jax.experimental.pallas import tpu_sc as plsc

_VOCAB = 1000000
_EMBED = 32
_HALF = _EMBED // 2
_WORDS = 50
_BATCH = 4096

_VC = 1024
_CH = 4
_BB = _VC * _CH
_TCNB = 245
_VPAD = _TCNB * _BB

_HIMASK = -65536


def _detile_body(x_ref, o_ref, scr):
    for c in range(_CH):
        for q in range(8):
            xq = x_ref[:, c * _VC + 128 * q: c * _VC + 128 * (q + 1)]
            xb = lax.bitcast_convert_type(
                xq.astype(jnp.bfloat16).astype(jnp.float32), jnp.int32)
            w = (lax.shift_right_logical(xb[0:_HALF, :], 16)
                 | (xb[_HALF:_EMBED, :] & jnp.int32(_HIMASK)))
            scr[c, :, _HALF * q:_HALF * (q + 1)] = w.T
        o_ref[pl.ds(c * _VC * _HALF, _VC * _HALF)] = (
            scr[c].reshape(_VC * _HALF))


def _detile(table_t):
    return pl.pallas_call(
        _detile_body,
        grid=(_TCNB,),
        in_specs=[pl.BlockSpec((_EMBED, _BB), lambda j: (0, j))],
        out_specs=pl.BlockSpec((_BB * _HALF,), lambda j: (j,)),
        out_shape=jax.ShapeDtypeStruct((_VPAD * _HALF,), jnp.int32),
        scratch_shapes=[pltpu.VMEM((_CH, 128, 128), jnp.int32)],
    )(table_t)


_NC, _NS = 2, 16
_NW = _NC * _NS
_BPW = _BATCH // _NW
_GL = 16
_PAIRB = 2 * _GL
_NP = _BPW // _PAIRB
_RPP = _PAIRB * _WORDS
_IPT = _NP * _RPP
_IDXW = 128
_ICH = 16


def _rsqrt(x):
    i = plsc.bitcast(x, jnp.int32)
    i = jnp.int32(0x5F3759DF) - lax.shift_right_logical(i, 1)
    y = plsc.bitcast(i, jnp.float32)
    for _ in range(3):
        y = y * (1.5 - 0.5 * x * y * y)
    return y


def _tile_body(vid_hbm, table_hbm, f0_hbm, f1_hbm, fb0_hbm, fb1_hbm, out_hbm,
               idx_f, rows_v, f0_v, f1_v, fb0_v, fb1_v, o0_v, o1_v,
               sem0, sem1):
    wid = lax.axis_index("s") * _NC + lax.axis_index("c")
    sems = (sem0, sem1)

    stage = [
        pltpu.async_copy(vid_hbm.at[pl.ds(wid * _IPT, _IPT)], idx_f, sem0),
        pltpu.async_copy(f0_hbm, f0_v, sem0),
        pltpu.async_copy(f1_hbm, f1_v, sem0),
        pltpu.async_copy(fb0_hbm, fb0_v, sem0),
        pltpu.async_copy(fb1_hbm, fb1_v, sem0),
    ]
    for c in stage:
        c.wait()

    def idx_body(i, carry):
        s = i * _ICH
        v = idx_f[pl.ds(s, _ICH)]
        rowid = ((v & jnp.int32(~1023))
                 + lax.shift_left(v & jnp.int32(127), 3)
                 + (lax.shift_right_logical(v, 7) & jnp.int32(7)))
        idx_f[pl.ds(s, _ICH)] = rowid
        return carry

    lax.fori_loop(0, _IPT // _ICH, idx_body, jnp.int32(0))

    lanes = lax.iota(jnp.int32, _GL)
    bias0 = fb0_v[...]
    bias1 = fb1_v[...]
    row_base = lanes * _WORDS
    for g in range(_BPW // _GL):
        o0_v[pl.ds(g * _GL, _GL)] = bias0
        o1_v[pl.ds(g * _GL, _GL)] = bias1

    def fire(p):
        buf = p % 2
        sem = sems[buf]
        copies = []
        dst = 0
        while dst < _RPP:
            n = min(_IDXW, _RPP - dst)
            copies.append(pltpu.async_copy(
                table_hbm.at[idx_f.at[pl.ds(p * _RPP + dst, n)]],
                rows_v.at[buf, pl.ds(dst, n)], sem))
            dst += n
        return copies

    pending = {0: fire(0)}
    for p in range(_NP):
        buf = p % 2
        for c in pending.pop(p):
            c.wait()
        if p + 1 < _NP:
            pending[p + 1] = fire(p + 1)

        def w_body(w, carry, _rb=row_base, _buf=buf, _p=p):
            rowA = _rb + w
            rowB = rowA + _GL * _WORDS
            cw = jnp.full((_GL,), w * _EMBED, jnp.int32)
            z = jnp.zeros((_GL,), jnp.float32)
            ssA, a0A, a1A = z, z, z
            ssB, a0B, a1B = z, z, z
            for jp in range(_HALF):
                colv = jnp.full((_GL,), jp, jnp.int32)
                cl = cw + jp
                ch = cw + (jp + _HALF)
                c0l = plsc.load_gather(f0_v, [cl])
                c1l = plsc.load_gather(f1_v, [cl])
                c0h = plsc.load_gather(f0_v, [ch])
                c1h = plsc.load_gather(f1_v, [ch])
                wA = plsc.load_gather(rows_v.at[_buf], [rowA, colv])
                wB = plsc.load_gather(rows_v.at[_buf], [rowB, colv])
                dAl = plsc.bitcast(lax.shift_left(wA, 16), jnp.float32)
                dAh = plsc.bitcast(wA & jnp.int32(_HIMASK), jnp.float32)
                dBl = plsc.bitcast(lax.shift_left(wB, 16), jnp.float32)
                dBh = plsc.bitcast(wB & jnp.int32(_HIMASK), jnp.float32)
                ssA = ssA + dAl * dAl + dAh * dAh
                a0A = a0A + dAl * c0l + dAh * c0h
                a1A = a1A + dAl * c1l + dAh * c1h
                ssB = ssB + dBl * dBl + dBh * dBh
                a0B = a0B + dBl * c0l + dBh * c0h
                a1B = a1B + dBl * c1l + dBh * c1h
            sA = _rsqrt(jnp.maximum(ssA, 1.0))
            sB = _rsqrt(jnp.maximum(ssB, 1.0))
            plsc.addupdate(o0_v.at[pl.ds(_p * _PAIRB, _GL)], sA * a0A)
            plsc.addupdate(o1_v.at[pl.ds(_p * _PAIRB, _GL)], sA * a1A)
            plsc.addupdate(o0_v.at[pl.ds(_p * _PAIRB + _GL, _GL)], sB * a0B)
            plsc.addupdate(o1_v.at[pl.ds(_p * _PAIRB + _GL, _GL)], sB * a1B)
            return carry

        lax.fori_loop(0, _WORDS, w_body, jnp.int32(0))

    pltpu.sync_copy(o0_v, out_hbm.at[0, pl.ds(wid * _BPW, _BPW)])
    pltpu.sync_copy(o1_v, out_hbm.at[1, pl.ds(wid * _BPW, _BPW)])


def _sc_logits(vid1d, table, f0, f1, fb0, fb1):
    mesh = plsc.VectorSubcoreMesh(core_axis_name="c", subcore_axis_name="s")
    return pl.kernel(
        _tile_body,
        out_type=jax.ShapeDtypeStruct((2, _BATCH), jnp.float32),
        mesh=mesh,
        compiler_params=pltpu.CompilerParams(
            needs_layout_passes=False, use_tc_tiling_on_sc=False),
        scratch_types=[
            pltpu.VMEM((_IPT,), jnp.int32),
            pltpu.VMEM((2, _RPP, _HALF), jnp.int32),
            pltpu.VMEM((_WORDS * _EMBED,), jnp.float32),
            pltpu.VMEM((_WORDS * _EMBED,), jnp.float32),
            pltpu.VMEM((_GL,), jnp.float32),
            pltpu.VMEM((_GL,), jnp.float32),
            pltpu.VMEM((_BPW,), jnp.float32),
            pltpu.VMEM((_BPW,), jnp.float32),
            pltpu.SemaphoreType.DMA,
            pltpu.SemaphoreType.DMA,
        ],
    )(vid1d, table, f0, f1, fb0, fb1)


@jax.jit
def _run(vocab_id, table, fc_w, fc_b):
    lin = _detile(table.T)
    table_lin = lin.reshape(_VPAD, _HALF)
    vid1d = vocab_id.reshape(-1)
    fb0 = jnp.full((_GL,), fc_b[0], jnp.float32)
    fb1 = jnp.full((_GL,), fc_b[1], jnp.float32)
    out2 = _sc_logits(vid1d, table_lin, fc_w[0], fc_w[1], fb0, fb1)
    return out2.T


def kernel(vocab_id, table, fc_w, fc_b):
    return _run(vocab_id, table, fc_w, fc_b)


# speedup vs baseline: 70.0340x; 1.0551x over previous
import jax
import jax.numpy as jnp
from jax import lax
from jax.experimental import pallas as pl
from jax.experimental.pallas import tpu as pltpu
from jax.experimental.pallas import tpu_sc as plsc

_VOCAB = 1000000
_EMBED = 32
_HALF = _EMBED // 2
_WORDS = 50
_BATCH = 4096

_VC = 1024
_CH = 8
_BB = _VC * _CH
_TCNB = 123
_VPAD = _TCNB * _BB

_HIMASK = -65536


def _detile_body(x_ref, o_ref, scr):
    for c in range(_CH):
        for q in range(8):
            xq = x_ref[:, c * _VC + 128 * q: c * _VC + 128 * (q + 1)]
            xb = lax.bitcast_convert_type(
                xq.astype(jnp.bfloat16).astype(jnp.float32), jnp.int32)
            w = (lax.shift_right_logical(xb[0:_HALF, :], 16)
                 | (xb[_HALF:_EMBED, :] & jnp.int32(_HIMASK)))
            scr[c, :, _HALF * q:_HALF * (q + 1)] = w.T
        o_ref[pl.ds(c * _VC * _HALF, _VC * _HALF)] = (
            scr[c].reshape(_VC * _HALF))


def _detile(table_t):
    return pl.pallas_call(
        _detile_body,
        grid=(_TCNB,),
        in_specs=[pl.BlockSpec((_EMBED, _BB), lambda j: (0, j))],
        out_specs=pl.BlockSpec((_BB * _HALF,), lambda j: (j,)),
        out_shape=jax.ShapeDtypeStruct((_VPAD * _HALF,), jnp.int32),
        scratch_shapes=[pltpu.VMEM((_CH, 128, 128), jnp.int32)],
    )(table_t)


_NC, _NS = 2, 16
_NW = _NC * _NS
_BPW = _BATCH // _NW
_GL = 16
_PAIRB = 2 * _GL
_NP = _BPW // _PAIRB
_RPP = _PAIRB * _WORDS
_IPT = _NP * _RPP
_IDXW = 128
_ICH = 16


def _rsqrt(x):
    i = plsc.bitcast(x, jnp.int32)
    i = jnp.int32(0x5F3759DF) - lax.shift_right_logical(i, 1)
    y = plsc.bitcast(i, jnp.float32)
    for _ in range(3):
        y = y * (1.5 - 0.5 * x * y * y)
    return y


def _tile_body(vid_hbm, table_hbm, f0_hbm, f1_hbm, fb0_hbm, fb1_hbm, out_hbm,
               idx_f, rows_v, f0_v, f1_v, fb0_v, fb1_v, o0_v, o1_v,
               sem0, sem1):
    wid = lax.axis_index("s") * _NC + lax.axis_index("c")
    sems = (sem0, sem1)

    stage = [
        pltpu.async_copy(vid_hbm.at[pl.ds(wid * _IPT, _IPT)], idx_f, sem0),
        pltpu.async_copy(f0_hbm, f0_v, sem0),
        pltpu.async_copy(f1_hbm, f1_v, sem0),
        pltpu.async_copy(fb0_hbm, fb0_v, sem0),
        pltpu.async_copy(fb1_hbm, fb1_v, sem0),
    ]
    for c in stage:
        c.wait()

    def idx_body(i, carry):
        s = i * _ICH
        v = idx_f[pl.ds(s, _ICH)]
        rowid = ((v & jnp.int32(~1023))
                 + lax.shift_left(v & jnp.int32(127), 3)
                 + (lax.shift_right_logical(v, 7) & jnp.int32(7)))
        idx_f[pl.ds(s, _ICH)] = rowid
        return carry

    lax.fori_loop(0, _IPT // _ICH, idx_body, jnp.int32(0))

    lanes = lax.iota(jnp.int32, _GL)
    bias0 = fb0_v[...]
    bias1 = fb1_v[...]
    row_base = lanes * _WORDS
    for g in range(_BPW // _GL):
        o0_v[pl.ds(g * _GL, _GL)] = bias0
        o1_v[pl.ds(g * _GL, _GL)] = bias1

    def fire(p):
        buf = p % 2
        sem = sems[buf]
        copies = []
        dst = 0
        while dst < _RPP:
            n = min(_IDXW, _RPP - dst)
            copies.append(pltpu.async_copy(
                table_hbm.at[idx_f.at[pl.ds(p * _RPP + dst, n)]],
                rows_v.at[buf, pl.ds(dst, n)], sem))
            dst += n
        return copies

    pending = {0: fire(0)}
    for p in range(_NP):
        buf = p % 2
        for c in pending.pop(p):
            c.wait()
        if p + 1 < _NP:
            pending[p + 1] = fire(p + 1)

        def w_body(w, carry, _rb=row_base, _buf=buf, _p=p):
            rowA = _rb + w
            rowB = rowA + _GL * _WORDS
            cw = jnp.full((_GL,), w * _EMBED, jnp.int32)
            z = jnp.zeros((_GL,), jnp.float32)
            ssA, a0A, a1A = z, z, z
            ssB, a0B, a1B = z, z, z
            for jp in range(_HALF):
                colv = jnp.full((_GL,), jp, jnp.int32)
                cl = cw + jp
                ch = cw + (jp + _HALF)
                c0l = plsc.load_gather(f0_v, [cl])
                c1l = plsc.load_gather(f1_v, [cl])
                c0h = plsc.load_gather(f0_v, [ch])
                c1h = plsc.load_gather(f1_v, [ch])
                wA = plsc.load_gather(rows_v.at[_buf], [rowA, colv])
                wB = plsc.load_gather(rows_v.at[_buf], [rowB, colv])
                dAl = plsc.bitcast(lax.shift_left(wA, 16), jnp.float32)
                dAh = plsc.bitcast(wA & jnp.int32(_HIMASK), jnp.float32)
                dBl = plsc.bitcast(lax.shift_left(wB, 16), jnp.float32)
                dBh = plsc.bitcast(wB & jnp.int32(_HIMASK), jnp.float32)
                ssA = ssA + dAl * dAl + dAh * dAh
                a0A = a0A + dAl * c0l + dAh * c0h
                a1A = a1A + dAl * c1l + dAh * c1h
                ssB = ssB + dBl * dBl + dBh * dBh
                a0B = a0B + dBl * c0l + dBh * c0h
                a1B = a1B + dBl * c1l + dBh * c1h
            sA = _rsqrt(jnp.maximum(ssA, 1.0))
            sB = _rsqrt(jnp.maximum(ssB, 1.0))
            plsc.addupdate(o0_v.at[pl.ds(_p * _PAIRB, _GL)], sA * a0A)
            plsc.addupdate(o1_v.at[pl.ds(_p * _PAIRB, _GL)], sA * a1A)
            plsc.addupdate(o0_v.at[pl.ds(_p * _PAIRB + _GL, _GL)], sB * a0B)
            plsc.addupdate(o1_v.at[pl.ds(_p * _PAIRB + _GL, _GL)], sB * a1B)
            return carry

        lax.fori_loop(0, _WORDS, w_body, jnp.int32(0))

    pltpu.sync_copy(o0_v, out_hbm.at[0, pl.ds(wid * _BPW, _BPW)])
    pltpu.sync_copy(o1_v, out_hbm.at[1, pl.ds(wid * _BPW, _BPW)])


def _sc_logits(vid1d, table, f0, f1, fb0, fb1):
    mesh = plsc.VectorSubcoreMesh(core_axis_name="c", subcore_axis_name="s")
    return pl.kernel(
        _tile_body,
        out_type=jax.ShapeDtypeStruct((2, _BATCH), jnp.float32),
        mesh=mesh,
        compiler_params=pltpu.CompilerParams(
            needs_layout_passes=False, use_tc_tiling_on_sc=False),
        scratch_types=[
            pltpu.VMEM((_IPT,), jnp.int32),
            pltpu.VMEM((2, _RPP, _HALF), jnp.int32),
            pltpu.VMEM((_WORDS * _EMBED,), jnp.float32),
            pltpu.VMEM((_WORDS * _EMBED,), jnp.float32),
            pltpu.VMEM((_GL,), jnp.float32),
            pltpu.VMEM((_GL,), jnp.float32),
            pltpu.VMEM((_BPW,), jnp.float32),
            pltpu.VMEM((_BPW,), jnp.float32),
            pltpu.SemaphoreType.DMA,
            pltpu.SemaphoreType.DMA,
        ],
    )(vid1d, table, f0, f1, fb0, fb1)


@jax.jit
def _run(vocab_id, table, fc_w, fc_b):
    lin = _detile(table.T)
    table_lin = lin.reshape(_VPAD, _HALF)
    vid1d = vocab_id.reshape(-1)
    fb0 = jnp.full((_GL,), fc_b[0], jnp.float32)
    fb1 = jnp.full((_GL,), fc_b[1], jnp.float32)
    out2 = _sc_logits(vid1d, table_lin, fc_w[0], fc_w[1], fb0, fb1)
    return out2.T


def kernel(vocab_id, table, fc_w, fc_b):
    return _run(vocab_id, table, fc_w, fc_b)


# speedup vs baseline: 70.7188x; 1.0098x over previous
import jax
import jax.numpy as jnp
from jax import lax
from jax.experimental import pallas as pl
from jax.experimental.pallas import tpu as pltpu
from jax.experimental.pallas import tpu_sc as plsc

_VOCAB = 1000000
_EMBED = 32
_HALF = _EMBED // 2
_WORDS = 50
_BATCH = 4096

_VC = 1024
_CH = 16
_BB = _VC * _CH
_TCNB = 62
_VPAD = _TCNB * _BB

_HIMASK = -65536


def _detile_body(x_ref, o_ref, scr):
    for c in range(_CH):
        for q in range(8):
            xq = x_ref[:, c * _VC + 128 * q: c * _VC + 128 * (q + 1)]
            xb = lax.bitcast_convert_type(
                xq.astype(jnp.bfloat16).astype(jnp.float32), jnp.int32)
            w = (lax.shift_right_logical(xb[0:_HALF, :], 16)
                 | (xb[_HALF:_EMBED, :] & jnp.int32(_HIMASK)))
            scr[c, :, _HALF * q:_HALF * (q + 1)] = w.T
        o_ref[pl.ds(c * _VC * _HALF, _VC * _HALF)] = (
            scr[c].reshape(_VC * _HALF))


def _detile(table_t):
    return pl.pallas_call(
        _detile_body,
        grid=(_TCNB,),
        in_specs=[pl.BlockSpec((_EMBED, _BB), lambda j: (0, j))],
        out_specs=pl.BlockSpec((_BB * _HALF,), lambda j: (j,)),
        out_shape=jax.ShapeDtypeStruct((_VPAD * _HALF,), jnp.int32),
        scratch_shapes=[pltpu.VMEM((_CH, 128, 128), jnp.int32)],
    )(table_t)


_NC, _NS = 2, 16
_NW = _NC * _NS
_BPW = _BATCH // _NW
_GL = 16
_PAIRB = 2 * _GL
_NP = _BPW // _PAIRB
_RPP = _PAIRB * _WORDS
_IPT = _NP * _RPP
_IDXW = 128
_ICH = 16


def _rsqrt(x):
    i = plsc.bitcast(x, jnp.int32)
    i = jnp.int32(0x5F3759DF) - lax.shift_right_logical(i, 1)
    y = plsc.bitcast(i, jnp.float32)
    for _ in range(3):
        y = y * (1.5 - 0.5 * x * y * y)
    return y


def _tile_body(vid_hbm, table_hbm, f0_hbm, f1_hbm, fb0_hbm, fb1_hbm, out_hbm,
               idx_f, rows_v, f0_v, f1_v, fb0_v, fb1_v, o0_v, o1_v,
               sem0, sem1):
    wid = lax.axis_index("s") * _NC + lax.axis_index("c")
    sems = (sem0, sem1)

    stage = [
        pltpu.async_copy(vid_hbm.at[pl.ds(wid * _IPT, _IPT)], idx_f, sem0),
        pltpu.async_copy(f0_hbm, f0_v, sem0),
        pltpu.async_copy(f1_hbm, f1_v, sem0),
        pltpu.async_copy(fb0_hbm, fb0_v, sem0),
        pltpu.async_copy(fb1_hbm, fb1_v, sem0),
    ]
    for c in stage:
        c.wait()

    def idx_body(i, carry):
        s = i * _ICH
        v = idx_f[pl.ds(s, _ICH)]
        rowid = ((v & jnp.int32(~1023))
                 + lax.shift_left(v & jnp.int32(127), 3)
                 + (lax.shift_right_logical(v, 7) & jnp.int32(7)))
        idx_f[pl.ds(s, _ICH)] = rowid
        return carry

    lax.fori_loop(0, _IPT // _ICH, idx_body, jnp.int32(0))

    lanes = lax.iota(jnp.int32, _GL)
    bias0 = fb0_v[...]
    bias1 = fb1_v[...]
    row_base = lanes * _WORDS
    for g in range(_BPW // _GL):
        o0_v[pl.ds(g * _GL, _GL)] = bias0
        o1_v[pl.ds(g * _GL, _GL)] = bias1

    def fire(p):
        buf = p % 2
        sem = sems[buf]
        copies = []
        dst = 0
        while dst < _RPP:
            n = min(_IDXW, _RPP - dst)
            copies.append(pltpu.async_copy(
                table_hbm.at[idx_f.at[pl.ds(p * _RPP + dst, n)]],
                rows_v.at[buf, pl.ds(dst, n)], sem))
            dst += n
        return copies

    pending = {0: fire(0)}
    for p in range(_NP):
        buf = p % 2
        for c in pending.pop(p):
            c.wait()
        if p + 1 < _NP:
            pending[p + 1] = fire(p + 1)

        def w_body(w, carry, _rb=row_base, _buf=buf, _p=p):
            rowA = _rb + w
            rowB = rowA + _GL * _WORDS
            cw = jnp.full((_GL,), w * _EMBED, jnp.int32)
            z = jnp.zeros((_GL,), jnp.float32)
            ssA, a0A, a1A = z, z, z
            ssB, a0B, a1B = z, z, z
            for jp in range(_HALF):
                colv = jnp.full((_GL,), jp, jnp.int32)
                cl = cw + jp
                ch = cw + (jp + _HALF)
                c0l = plsc.load_gather(f0_v, [cl])
                c1l = plsc.load_gather(f1_v, [cl])
                c0h = plsc.load_gather(f0_v, [ch])
                c1h = plsc.load_gather(f1_v, [ch])
                wA = plsc.load_gather(rows_v.at[_buf], [rowA, colv])
                wB = plsc.load_gather(rows_v.at[_buf], [rowB, colv])
                dAl = plsc.bitcast(lax.shift_left(wA, 16), jnp.float32)
                dAh = plsc.bitcast(wA & jnp.int32(_HIMASK), jnp.float32)
                dBl = plsc.bitcast(lax.shift_left(wB, 16), jnp.float32)
                dBh = plsc.bitcast(wB & jnp.int32(_HIMASK), jnp.float32)
                ssA = ssA + dAl * dAl + dAh * dAh
                a0A = a0A + dAl * c0l + dAh * c0h
                a1A = a1A + dAl * c1l + dAh * c1h
                ssB = ssB + dBl * dBl + dBh * dBh
                a0B = a0B + dBl * c0l + dBh * c0h
                a1B = a1B + dBl * c1l + dBh * c1h
            sA = _rsqrt(jnp.maximum(ssA, 1.0))
            sB = _rsqrt(jnp.maximum(ssB, 1.0))
            plsc.addupdate(o0_v.at[pl.ds(_p * _PAIRB, _GL)], sA * a0A)
            plsc.addupdate(o1_v.at[pl.ds(_p * _PAIRB, _GL)], sA * a1A)
            plsc.addupdate(o0_v.at[pl.ds(_p * _PAIRB + _GL, _GL)], sB * a0B)
            plsc.addupdate(o1_v.at[pl.ds(_p * _PAIRB + _GL, _GL)], sB * a1B)
            return carry

        lax.fori_loop(0, _WORDS, w_body, jnp.int32(0))

    pltpu.sync_copy(o0_v, out_hbm.at[0, pl.ds(wid * _BPW, _BPW)])
    pltpu.sync_copy(o1_v, out_hbm.at[1, pl.ds(wid * _BPW, _BPW)])


def _sc_logits(vid1d, table, f0, f1, fb0, fb1):
    mesh = plsc.VectorSubcoreMesh(core_axis_name="c", subcore_axis_name="s")
    return pl.kernel(
        _tile_body,
        out_type=jax.ShapeDtypeStruct((2, _BATCH), jnp.float32),
        mesh=mesh,
        compiler_params=pltpu.CompilerParams(
            needs_layout_passes=False, use_tc_tiling_on_sc=False),
        scratch_types=[
            pltpu.VMEM((_IPT,), jnp.int32),
            pltpu.VMEM((2, _RPP, _HALF), jnp.int32),
            pltpu.VMEM((_WORDS * _EMBED,), jnp.float32),
            pltpu.VMEM((_WORDS * _EMBED,), jnp.float32),
            pltpu.VMEM((_GL,), jnp.float32),
            pltpu.VMEM((_GL,), jnp.float32),
            pltpu.VMEM((_BPW,), jnp.float32),
            pltpu.VMEM((_BPW,), jnp.float32),
            pltpu.SemaphoreType.DMA,
            pltpu.SemaphoreType.DMA,
        ],
    )(vid1d, table, f0, f1, fb0, fb1)


@jax.jit
def _run(vocab_id, table, fc_w, fc_b):
    lin = _detile(table.T)
    table_lin = lin.reshape(_VPAD, _HALF)
    vid1d = vocab_id.reshape(-1)
    fb0 = jnp.full((_GL,), fc_b[0], jnp.float32)
    fb1 = jnp.full((_GL,), fc_b[1], jnp.float32)
    out2 = _sc_logits(vid1d, table_lin, fc_w[0], fc_w[1], fb0, fb1)
    return out2.T


def kernel(vocab_id, table, fc_w, fc_b):
    return _run(vocab_id, table, fc_w, fc_b)
